# Initial kernel scaffold; baseline (speedup 1.0000x reference)
import functools

import jax
import jax.numpy as jnp
from jax import lax
from jax.experimental import pallas as pl
from jax.experimental.pallas import tpu as pltpu
from jax.experimental.pallas import tpu_sc as plsc

N = 10000
E = 320000
D_FEAT = 128
NCLASS = 64
ALPHA = 0.1
K = 10

NC = 2
NS = 16
HALF = NCLASS // NC
CHUNK = 128
E_TILE = E // NS
NCH = -(-E_TILE // CHUNK)
EP_TILE = NCH * CHUNK
ROWS_TILE = N // NS
MROW_BLK = 1000


def _mlp_body(x_ref, w1_ref, b1_ref, w2_ref, b2_ref, z0_ref, h0a_ref):
    h = jnp.maximum(
        jax.lax.dot_general(x_ref[...], w1_ref[...], (((1,), (1,)), ((), ())),
                            preferred_element_type=jnp.float32) + b1_ref[...],
        0.0)
    z = jax.lax.dot_general(h, w2_ref[...], (((1,), (1,)), ((), ())),
                            preferred_element_type=jnp.float32) + b2_ref[...]
    z0_ref[0] = z[:, :HALF]
    z0_ref[1] = z[:, HALF:]
    h0a_ref[0] = ALPHA * z[:, :HALF]
    h0a_ref[1] = ALPHA * z[:, HALF:]


def _lsm_body(z_ref, o_ref):
    z = z_ref[...]
    m = jnp.max(z, axis=1, keepdims=True)
    e = jnp.exp(z - m)
    s = jnp.sum(e, axis=1, keepdims=True)
    o_ref[...] = z - m - jnp.log(s)


def _sc_body(z0_hbm, h0a_hbm, col_hbm, row_hbm, val_hbm, out_hbm,
             col_v, row_v, val_v, rows_v, h0a_t, zb0, zb1, gsem):
    c = lax.axis_index("c")
    s = lax.axis_index("s")
    r0 = s * ROWS_TILE

    pltpu.sync_copy(col_hbm.at[s], col_v)
    pltpu.sync_copy(row_hbm.at[s], row_v)
    pltpu.sync_copy(val_hbm.at[s], val_v)

    @pl.loop(0, EP_TILE // 16)
    def _scale(i):
        val_v[pl.ds(i * 16, 16)] = val_v[pl.ds(i * 16, 16)] * (1.0 - ALPHA)

    pltpu.sync_copy(z0_hbm.at[c, pl.ds(r0, ROWS_TILE)], h0a_t)
    pltpu.sync_copy(h0a_t, zb0.at[pl.ds(r0, ROWS_TILE)])
    pltpu.sync_copy(h0a_hbm.at[c, pl.ds(r0, ROWS_TILE)], h0a_t)
    plsc.subcore_barrier()

    for k in range(K):
        zr, zw = (zb0, zb1) if k % 2 == 0 else (zb1, zb0)
        pltpu.sync_copy(h0a_t, zw.at[pl.ds(r0, ROWS_TILE)])
        plsc.subcore_barrier()

        @pl.loop(0, NCH)
        def _chunk(j):
            pltpu.async_copy(zr.at[col_v.at[j]], rows_v, gsem).wait()

            @pl.loop(0, CHUNK)
            def _edge(e):
                vsp = plsc.load_gather(
                    val_v, [jnp.full((16,), j * CHUNK + e, jnp.int32)])
                rows_v[e, pl.ds(0, 16)] = rows_v[e, pl.ds(0, 16)] * vsp
                rows_v[e, pl.ds(16, 16)] = rows_v[e, pl.ds(16, 16)] * vsp

            pltpu.sync_copy(rows_v, zw.at[row_v.at[j]], add=True)

        plsc.subcore_barrier()

    pltpu.sync_copy(zb0.at[pl.ds(r0, ROWS_TILE)],
                    out_hbm.at[c, pl.ds(r0, ROWS_TILE)])


def kernel(x, edge_index, values, W1, b1, W2, b2):
    x = x.astype(jnp.float32)

    z0h, h0a = pl.pallas_call(
        _mlp_body,
        grid=(N // MROW_BLK,),
        in_specs=[
            pl.BlockSpec((MROW_BLK, D_FEAT), lambda i: (i, 0)),
            pl.BlockSpec((NCLASS, D_FEAT), lambda i: (0, 0)),
            pl.BlockSpec((1, NCLASS), lambda i: (0, 0)),
            pl.BlockSpec((NCLASS, NCLASS), lambda i: (0, 0)),
            pl.BlockSpec((1, NCLASS), lambda i: (0, 0)),
        ],
        out_specs=[
            pl.BlockSpec((NC, MROW_BLK, HALF), lambda i: (0, i, 0)),
            pl.BlockSpec((NC, MROW_BLK, HALF), lambda i: (0, i, 0)),
        ],
        out_shape=[
            jax.ShapeDtypeStruct((NC, N, HALF), jnp.float32),
            jax.ShapeDtypeStruct((NC, N, HALF), jnp.float32),
        ],
    )(x, W1, b1.reshape(1, NCLASS), W2, b2.reshape(1, NCLASS))

    row = edge_index[0].astype(jnp.int32)
    col = edge_index[1].astype(jnp.int32)
    pad = NS * EP_TILE - E
    row_p = jnp.pad(row, (0, pad)).reshape(NS, NCH, CHUNK)
    col_p = jnp.pad(col, (0, pad)).reshape(NS, NCH, CHUNK)
    val_p = jnp.pad(values.astype(jnp.float32), (0, pad)).reshape(NS, EP_TILE)

    mesh = plsc.VectorSubcoreMesh(core_axis_name="c", subcore_axis_name="s")
    zf = pl.kernel(
        _sc_body,
        out_type=jax.ShapeDtypeStruct((NC, N, HALF), jnp.float32),
        mesh=mesh,
        scratch_types=[
            pltpu.VMEM((NCH, CHUNK), jnp.int32),
            pltpu.VMEM((NCH, CHUNK), jnp.int32),
            pltpu.VMEM((EP_TILE,), jnp.float32),
            pltpu.VMEM((CHUNK, HALF), jnp.float32),
            pltpu.VMEM((ROWS_TILE, HALF), jnp.float32),
            pltpu.VMEM_SHARED((N, HALF), jnp.float32),
            pltpu.VMEM_SHARED((N, HALF), jnp.float32),
            pltpu.SemaphoreType.DMA,
        ],
    )(z0h, h0a, col_p, row_p, val_p)

    zcat = jnp.concatenate([zf[0], zf[1]], axis=1)

    out = pl.pallas_call(
        _lsm_body,
        grid=(N // MROW_BLK,),
        in_specs=[pl.BlockSpec((MROW_BLK, NCLASS), lambda i: (i, 0))],
        out_specs=pl.BlockSpec((MROW_BLK, NCLASS), lambda i: (i, 0)),
        out_shape=jax.ShapeDtypeStruct((N, NCLASS), jnp.float32),
    )(zcat)
    return out


# speedup vs baseline: 12.2493x; 12.2493x over previous
import functools

import jax
import jax.numpy as jnp
from jax import lax
from jax.experimental import pallas as pl
from jax.experimental.pallas import tpu as pltpu
from jax.experimental.pallas import tpu_sc as plsc

N = 10000
E = 320000
D_FEAT = 128
NCLASS = 64
ALPHA = 0.1
K = 10

NC = 2
NS = 16
HALF = NCLASS // NC
CHUNK = 128
E_TILE = E // NS
NCH = -(-E_TILE // CHUNK)
EP_TILE = NCH * CHUNK
NP_ = 10240
ROWS_TILE = NP_ // NS
MROW_BLK = 1000


def _mlp_body(x_ref, w1_ref, b1_ref, w2_ref, b2_ref, z0_ref, h0a_ref):
    h = jnp.maximum(
        jax.lax.dot_general(x_ref[...], w1_ref[...], (((1,), (1,)), ((), ())),
                            preferred_element_type=jnp.float32) + b1_ref[...],
        0.0)
    z = jax.lax.dot_general(h, w2_ref[...], (((1,), (1,)), ((), ())),
                            preferred_element_type=jnp.float32) + b2_ref[...]
    z0_ref[0] = z[:, :HALF]
    z0_ref[1] = z[:, HALF:]
    h0a_ref[0] = ALPHA * z[:, :HALF]
    h0a_ref[1] = ALPHA * z[:, HALF:]


def _lsm_body(z_ref, o_ref):
    z = z_ref[...]
    m = jnp.max(z, axis=1, keepdims=True)
    e = jnp.exp(z - m)
    s = jnp.sum(e, axis=1, keepdims=True)
    o_ref[...] = z - m - jnp.log(s)


def _sc_body(z0_hbm, h0a_hbm, col_hbm, row_hbm, val_hbm, out_hbm,
             col_v, row_v, val_v, rows_v, h0a_t, zb0, zb1, gsem):
    c = lax.axis_index("c")
    s = lax.axis_index("s")
    r0 = s * ROWS_TILE

    pltpu.sync_copy(col_hbm.at[s], col_v)
    pltpu.sync_copy(row_hbm.at[s], row_v)
    pltpu.sync_copy(val_hbm.at[s], val_v)

    @pl.loop(0, EP_TILE // 16)
    def _scale(i):
        j, g = i // (CHUNK // 16), i % (CHUNK // 16)
        val_v[j, pl.ds(g * 16, 16)] = val_v[j, pl.ds(g * 16, 16)] * (1.0 - ALPHA)

    pltpu.sync_copy(z0_hbm.at[c, pl.ds(r0, ROWS_TILE)], h0a_t)
    pltpu.sync_copy(h0a_t, zb0.at[pl.ds(r0, ROWS_TILE)])
    pltpu.sync_copy(h0a_hbm.at[c, pl.ds(r0, ROWS_TILE)], h0a_t)
    plsc.subcore_barrier()

    for k in range(K):
        zr, zw = (zb0, zb1) if k % 2 == 0 else (zb1, zb0)
        pltpu.sync_copy(h0a_t, zw.at[pl.ds(r0, ROWS_TILE)])
        plsc.subcore_barrier()

        @pl.loop(0, NCH)
        def _chunk(j):
            pltpu.async_copy(zr.at[col_v.at[j]], rows_v, gsem).wait()

            @pl.loop(0, CHUNK // 16)
            def _grp(g):
                vals16 = val_v[j, pl.ds(g * 16, 16)]
                for i in range(16):
                    e = g * 16 + i
                    vsp = jnp.full((16,), vals16[i], jnp.float32)
                    rows_v[e, pl.ds(0, 16)] = rows_v[e, pl.ds(0, 16)] * vsp
                    rows_v[e, pl.ds(16, 16)] = rows_v[e, pl.ds(16, 16)] * vsp

            pltpu.sync_copy(rows_v, zw.at[row_v.at[j]], add=True)

        plsc.subcore_barrier()

    pltpu.sync_copy(zb0.at[pl.ds(r0, ROWS_TILE)],
                    out_hbm.at[c, pl.ds(r0, ROWS_TILE)])


def kernel(x, edge_index, values, W1, b1, W2, b2):
    x = x.astype(jnp.float32)

    z0h, h0a = pl.pallas_call(
        _mlp_body,
        grid=(N // MROW_BLK,),
        in_specs=[
            pl.BlockSpec((MROW_BLK, D_FEAT), lambda i: (i, 0)),
            pl.BlockSpec((NCLASS, D_FEAT), lambda i: (0, 0)),
            pl.BlockSpec((1, NCLASS), lambda i: (0, 0)),
            pl.BlockSpec((NCLASS, NCLASS), lambda i: (0, 0)),
            pl.BlockSpec((1, NCLASS), lambda i: (0, 0)),
        ],
        out_specs=[
            pl.BlockSpec((NC, MROW_BLK, HALF), lambda i: (0, i, 0)),
            pl.BlockSpec((NC, MROW_BLK, HALF), lambda i: (0, i, 0)),
        ],
        out_shape=[
            jax.ShapeDtypeStruct((NC, N, HALF), jnp.float32),
            jax.ShapeDtypeStruct((NC, N, HALF), jnp.float32),
        ],
    )(x, W1, b1.reshape(1, NCLASS), W2, b2.reshape(1, NCLASS))

    row = edge_index[0].astype(jnp.int32)
    col = edge_index[1].astype(jnp.int32)
    pad = NS * EP_TILE - E
    row_p = jnp.pad(row, (0, pad)).reshape(NS, NCH, CHUNK)
    col_p = jnp.pad(col, (0, pad)).reshape(NS, NCH, CHUNK)
    val_p = jnp.pad(values.astype(jnp.float32), (0, pad)).reshape(NS, NCH, CHUNK)

    z0p = jnp.pad(z0h, ((0, 0), (0, NP_ - N), (0, 0)))
    h0p = jnp.pad(h0a, ((0, 0), (0, NP_ - N), (0, 0)))

    mesh = plsc.VectorSubcoreMesh(core_axis_name="c", subcore_axis_name="s")
    zf = pl.kernel(
        _sc_body,
        out_type=jax.ShapeDtypeStruct((NC, NP_, HALF), jnp.float32),
        mesh=mesh,
        compiler_params=pltpu.CompilerParams(use_tc_tiling_on_sc=False),
        scratch_types=[
            pltpu.VMEM((NCH, CHUNK), jnp.int32),
            pltpu.VMEM((NCH, CHUNK), jnp.int32),
            pltpu.VMEM((NCH, CHUNK), jnp.float32),
            pltpu.VMEM((CHUNK, HALF), jnp.float32),
            pltpu.VMEM((ROWS_TILE, HALF), jnp.float32),
            pltpu.VMEM_SHARED((NP_, HALF), jnp.float32),
            pltpu.VMEM_SHARED((NP_, HALF), jnp.float32),
            pltpu.SemaphoreType.DMA,
        ],
    )(z0p, h0p, col_p, row_p, val_p)

    zcat = jnp.concatenate([zf[0, :N], zf[1, :N]], axis=1)

    out = pl.pallas_call(
        _lsm_body,
        grid=(N // MROW_BLK,),
        in_specs=[pl.BlockSpec((MROW_BLK, NCLASS), lambda i: (i, 0))],
        out_specs=pl.BlockSpec((MROW_BLK, NCLASS), lambda i: (i, 0)),
        out_shape=jax.ShapeDtypeStruct((N, NCLASS), jnp.float32),
    )(zcat)
    return out


# speedup vs baseline: 18.2961x; 1.4936x over previous
import functools

import jax
import jax.numpy as jnp
from jax import lax
from jax.experimental import pallas as pl
from jax.experimental.pallas import tpu as pltpu
from jax.experimental.pallas import tpu_sc as plsc

N = 10000
E = 320000
D_FEAT = 128
NCLASS = 64
ALPHA = 0.1
K = 10

NC = 2
NS = 16
HALF = NCLASS // NC
CHUNK = 128
E_TILE = E // NS
NCH = 158
EP_TILE = NCH * CHUNK
NP_ = 10240
ROWS_TILE = NP_ // NS
MROW_BLK = 1000


def _mlp_body(x_ref, w1_ref, b1_ref, w2_ref, b2_ref, z0_ref, h0a_ref):
    h = jnp.maximum(
        jax.lax.dot_general(x_ref[...], w1_ref[...], (((1,), (1,)), ((), ())),
                            preferred_element_type=jnp.float32) + b1_ref[...],
        0.0)
    z = jax.lax.dot_general(h, w2_ref[...], (((1,), (1,)), ((), ())),
                            preferred_element_type=jnp.float32) + b2_ref[...]
    z0_ref[0] = z[:, :HALF]
    z0_ref[1] = z[:, HALF:]
    h0a_ref[0] = ALPHA * z[:, :HALF]
    h0a_ref[1] = ALPHA * z[:, HALF:]


def _lsm_body(z_ref, o_ref):
    z = z_ref[...]
    m = jnp.max(z, axis=1, keepdims=True)
    e = jnp.exp(z - m)
    s = jnp.sum(e, axis=1, keepdims=True)
    o_ref[...] = z - m - jnp.log(s)


def _sc_body(z0_hbm, h0a_hbm, col_hbm, row_hbm, val_hbm, out_hbm,
             col_v, row_v, val_v, gb0, gb1, sb0, sb1, zb0, zb1,
             gsem0, gsem1, ssem0, ssem1):
    c = lax.axis_index("c")
    s = lax.axis_index("s")
    r0 = s * ROWS_TILE

    pltpu.sync_copy(col_hbm.at[s], col_v)
    pltpu.sync_copy(row_hbm.at[s], row_v)
    pltpu.sync_copy(val_hbm.at[s], val_v)

    @pl.loop(0, EP_TILE // 16)
    def _scale(i):
        j, g = i // (CHUNK // 16), i % (CHUNK // 16)
        val_v[j, pl.ds(g * 16, 16)] = val_v[j, pl.ds(g * 16, 16)] * (1.0 - ALPHA)

    pltpu.sync_copy(z0_hbm.at[c, pl.ds(r0, ROWS_TILE)],
                    zb0.at[pl.ds(r0, ROWS_TILE)])
    plsc.subcore_barrier()

    def step(zr, zw):
        pltpu.sync_copy(h0a_hbm.at[c, pl.ds(r0, ROWS_TILE)],
                        zw.at[pl.ds(r0, ROWS_TILE)])
        plsc.subcore_barrier()

        def gissue(j, gbuf, gsem):
            pltpu.async_copy(zr.at[col_v.at[j]], gbuf, gsem)

        def gwait(j, gbuf, gsem):
            pltpu.make_async_copy(zr.at[col_v.at[j]], gbuf, gsem).wait()

        def mult(j, gbuf, sbuf):
            @pl.loop(0, CHUNK // 16)
            def _grp(g):
                vals16 = val_v[j, pl.ds(g * 16, 16)]
                for i in range(16):
                    e = g * 16 + i
                    vsp = jnp.full((16,), vals16[i], jnp.float32)
                    sbuf[e, pl.ds(0, 16)] = gbuf[e, pl.ds(0, 16)] * vsp
                    sbuf[e, pl.ds(16, 16)] = gbuf[e, pl.ds(16, 16)] * vsp

        def sissue(j, sbuf, ssem):
            pltpu.async_copy(sbuf, zw.at[row_v.at[j]], ssem, add=True)

        def swait(j, sbuf, ssem):
            pltpu.make_async_copy(sbuf, zw.at[row_v.at[j]], ssem).wait()

        gissue(0, gb0, gsem0)
        gwait(0, gb0, gsem0); gissue(1, gb1, gsem1)
        mult(0, gb0, sb0); sissue(0, sb0, ssem0)
        gwait(1, gb1, gsem1); gissue(2, gb0, gsem0)
        mult(1, gb1, sb1); sissue(1, sb1, ssem1)

        @pl.loop(0, (NCH - 4) // 2)
        def _pair(p):
            j = 2 + 2 * p
            gwait(j, gb0, gsem0); gissue(j + 1, gb1, gsem1)
            swait(j - 2, sb0, ssem0)
            mult(j, gb0, sb0); sissue(j, sb0, ssem0)
            j2 = j + 1
            gwait(j2, gb1, gsem1); gissue(j2 + 1, gb0, gsem0)
            swait(j2 - 2, sb1, ssem1)
            mult(j2, gb1, sb1); sissue(j2, sb1, ssem1)

        j = NCH - 2
        gwait(j, gb0, gsem0); gissue(NCH - 1, gb1, gsem1)
        swait(j - 2, sb0, ssem0)
        mult(j, gb0, sb0); sissue(j, sb0, ssem0)
        j2 = NCH - 1
        gwait(j2, gb1, gsem1)
        swait(j2 - 2, sb1, ssem1)
        mult(j2, gb1, sb1); sissue(j2, sb1, ssem1)
        swait(NCH - 2, sb0, ssem0)
        swait(NCH - 1, sb1, ssem1)
        plsc.subcore_barrier()

    for k in range(K // 2):
        step(zb0, zb1)
        step(zb1, zb0)

    pltpu.sync_copy(zb0.at[pl.ds(r0, ROWS_TILE)],
                    out_hbm.at[c, pl.ds(r0, ROWS_TILE)])


def kernel(x, edge_index, values, W1, b1, W2, b2):
    x = x.astype(jnp.float32)

    z0h, h0a = pl.pallas_call(
        _mlp_body,
        grid=(N // MROW_BLK,),
        in_specs=[
            pl.BlockSpec((MROW_BLK, D_FEAT), lambda i: (i, 0)),
            pl.BlockSpec((NCLASS, D_FEAT), lambda i: (0, 0)),
            pl.BlockSpec((1, NCLASS), lambda i: (0, 0)),
            pl.BlockSpec((NCLASS, NCLASS), lambda i: (0, 0)),
            pl.BlockSpec((1, NCLASS), lambda i: (0, 0)),
        ],
        out_specs=[
            pl.BlockSpec((NC, MROW_BLK, HALF), lambda i: (0, i, 0)),
            pl.BlockSpec((NC, MROW_BLK, HALF), lambda i: (0, i, 0)),
        ],
        out_shape=[
            jax.ShapeDtypeStruct((NC, N, HALF), jnp.float32),
            jax.ShapeDtypeStruct((NC, N, HALF), jnp.float32),
        ],
    )(x, W1, b1.reshape(1, NCLASS), W2, b2.reshape(1, NCLASS))

    row = edge_index[0].astype(jnp.int32)
    col = edge_index[1].astype(jnp.int32)
    pad = NS * EP_TILE - E
    row_p = jnp.pad(row, (0, pad)).reshape(NS, NCH, CHUNK)
    col_p = jnp.pad(col, (0, pad)).reshape(NS, NCH, CHUNK)
    val_p = jnp.pad(values.astype(jnp.float32), (0, pad)).reshape(NS, NCH, CHUNK)

    z0p = jnp.pad(z0h, ((0, 0), (0, NP_ - N), (0, 0)))
    h0p = jnp.pad(h0a, ((0, 0), (0, NP_ - N), (0, 0)))

    mesh = plsc.VectorSubcoreMesh(core_axis_name="c", subcore_axis_name="s")
    zf = pl.kernel(
        _sc_body,
        out_type=jax.ShapeDtypeStruct((NC, NP_, HALF), jnp.float32),
        mesh=mesh,
        compiler_params=pltpu.CompilerParams(use_tc_tiling_on_sc=False),
        scratch_types=[
            pltpu.VMEM((NCH, CHUNK), jnp.int32),
            pltpu.VMEM((NCH, CHUNK), jnp.int32),
            pltpu.VMEM((NCH, CHUNK), jnp.float32),
            pltpu.VMEM((CHUNK, HALF), jnp.float32),
            pltpu.VMEM((CHUNK, HALF), jnp.float32),
            pltpu.VMEM((CHUNK, HALF), jnp.float32),
            pltpu.VMEM((CHUNK, HALF), jnp.float32),
            pltpu.VMEM_SHARED((NP_, HALF), jnp.float32),
            pltpu.VMEM_SHARED((NP_, HALF), jnp.float32),
            pltpu.SemaphoreType.DMA,
            pltpu.SemaphoreType.DMA,
            pltpu.SemaphoreType.DMA,
            pltpu.SemaphoreType.DMA,
        ],
    )(z0p, h0p, col_p, row_p, val_p)

    zcat = jnp.concatenate([zf[0, :N], zf[1, :N]], axis=1)

    out = pl.pallas_call(
        _lsm_body,
        grid=(N // MROW_BLK,),
        in_specs=[pl.BlockSpec((MROW_BLK, NCLASS), lambda i: (i, 0))],
        out_specs=pl.BlockSpec((MROW_BLK, NCLASS), lambda i: (i, 0)),
        out_shape=jax.ShapeDtypeStruct((N, NCLASS), jnp.float32),
    )(zcat)
    return out


# speedup vs baseline: 31.2138x; 1.7060x over previous
import functools

import jax
import jax.numpy as jnp
from jax import lax
from jax.experimental import pallas as pl
from jax.experimental.pallas import tpu as pltpu
from jax.experimental.pallas import tpu_sc as plsc

N = 10000
E = 320000
D_FEAT = 128
NCLASS = 64
ALPHA = 0.1
K = 10

NC = 2
NS = 16
HALF = NCLASS // NC
CHUNK = 128
E_TILE = E // NS
NCH = 158
EP_TILE = NCH * CHUNK
NP_ = 10240
ROWS_TILE = NP_ // NS
MROW_BLK = 2000


def _mlp_body(x_ref, w1_ref, b1_ref, w2_ref, b2_ref, z0_ref, h0a_ref):
    h = jnp.maximum(
        jax.lax.dot_general(x_ref[...], w1_ref[...], (((1,), (1,)), ((), ())),
                            preferred_element_type=jnp.float32) + b1_ref[...],
        0.0)
    z = jax.lax.dot_general(h, w2_ref[...], (((1,), (1,)), ((), ())),
                            preferred_element_type=jnp.float32) + b2_ref[...]
    z0_ref[0] = z[:, :HALF].astype(jnp.bfloat16)
    z0_ref[1] = z[:, HALF:].astype(jnp.bfloat16)
    h0a_ref[0] = (ALPHA * z[:, :HALF]).astype(jnp.bfloat16)
    h0a_ref[1] = (ALPHA * z[:, HALF:]).astype(jnp.bfloat16)


def _lsm_body(z_ref, o_ref):
    z = z_ref[...].astype(jnp.float32)
    m = jnp.max(z, axis=1, keepdims=True)
    e = jnp.exp(z - m)
    s = jnp.sum(e, axis=1, keepdims=True)
    o_ref[...] = z - m - jnp.log(s)


def _sc_body(z0_hbm, h0a_hbm, col_hbm, row_hbm, val_hbm, out_hbm,
             col_v, row_v, val_v, gb0, gb1, sb0, sb1, zb0, zb1,
             gsem0, gsem1, ssem0, ssem1):
    c = lax.axis_index("c")
    s = lax.axis_index("s")
    r0 = s * ROWS_TILE

    pltpu.sync_copy(col_hbm.at[s], col_v)
    pltpu.sync_copy(row_hbm.at[s], row_v)
    pltpu.sync_copy(val_hbm.at[s], val_v)

    @pl.loop(0, EP_TILE // 16)
    def _scale(i):
        j, g = i // (CHUNK // 16), i % (CHUNK // 16)
        val_v[j, pl.ds(g * 16, 16)] = val_v[j, pl.ds(g * 16, 16)] * (1.0 - ALPHA)

    pltpu.sync_copy(z0_hbm.at[c, pl.ds(r0, ROWS_TILE)],
                    zb0.at[pl.ds(r0, ROWS_TILE)])
    plsc.subcore_barrier()

    def step(zr, zw):
        pltpu.sync_copy(h0a_hbm.at[c, pl.ds(r0, ROWS_TILE)],
                        zw.at[pl.ds(r0, ROWS_TILE)])
        plsc.subcore_barrier()

        def gissue(j, gbuf, gsem):
            pltpu.async_copy(zr.at[col_v.at[j]], gbuf, gsem)

        def gwait(j, gbuf, gsem):
            pltpu.make_async_copy(zr.at[col_v.at[j]], gbuf, gsem).wait()

        def mult(j, gbuf, sbuf):
            @pl.loop(0, CHUNK // 16)
            def _grp(g):
                vals16 = val_v[j, pl.ds(g * 16, 16)]
                for i in range(16):
                    e = g * 16 + i
                    vsp = jnp.full((16,), vals16[i], jnp.float32)
                    vspb = plsc.pack(vsp, vsp,
                                     format=plsc.PackFormat.INTERLEAVED)
                    sbuf[e, pl.ds(0, HALF)] = gbuf[e, pl.ds(0, HALF)] * vspb

        def sissue(j, sbuf, ssem):
            pltpu.async_copy(sbuf, zw.at[row_v.at[j]], ssem, add=True)

        def swait(j, sbuf, ssem):
            pltpu.make_async_copy(sbuf, zw.at[row_v.at[j]], ssem).wait()

        gissue(0, gb0, gsem0)
        gwait(0, gb0, gsem0); gissue(1, gb1, gsem1)
        mult(0, gb0, sb0); sissue(0, sb0, ssem0)
        gwait(1, gb1, gsem1); gissue(2, gb0, gsem0)
        mult(1, gb1, sb1); sissue(1, sb1, ssem1)

        @pl.loop(0, (NCH - 4) // 2)
        def _pair(p):
            j = 2 + 2 * p
            gwait(j, gb0, gsem0); gissue(j + 1, gb1, gsem1)
            swait(j - 2, sb0, ssem0)
            mult(j, gb0, sb0); sissue(j, sb0, ssem0)
            j2 = j + 1
            gwait(j2, gb1, gsem1); gissue(j2 + 1, gb0, gsem0)
            swait(j2 - 2, sb1, ssem1)
            mult(j2, gb1, sb1); sissue(j2, sb1, ssem1)

        j = NCH - 2
        gwait(j, gb0, gsem0); gissue(NCH - 1, gb1, gsem1)
        swait(j - 2, sb0, ssem0)
        mult(j, gb0, sb0); sissue(j, sb0, ssem0)
        j2 = NCH - 1
        gwait(j2, gb1, gsem1)
        swait(j2 - 2, sb1, ssem1)
        mult(j2, gb1, sb1); sissue(j2, sb1, ssem1)
        swait(NCH - 2, sb0, ssem0)
        swait(NCH - 1, sb1, ssem1)
        plsc.subcore_barrier()

    for k in range(K // 2):
        step(zb0, zb1)
        step(zb1, zb0)

    pltpu.sync_copy(zb0.at[pl.ds(r0, ROWS_TILE)],
                    out_hbm.at[c, pl.ds(r0, ROWS_TILE)])


def kernel(x, edge_index, values, W1, b1, W2, b2):
    x = x.astype(jnp.float32)

    z0h, h0a = pl.pallas_call(
        _mlp_body,
        grid=(N // MROW_BLK,),
        in_specs=[
            pl.BlockSpec((MROW_BLK, D_FEAT), lambda i: (i, 0)),
            pl.BlockSpec((NCLASS, D_FEAT), lambda i: (0, 0)),
            pl.BlockSpec((1, NCLASS), lambda i: (0, 0)),
            pl.BlockSpec((NCLASS, NCLASS), lambda i: (0, 0)),
            pl.BlockSpec((1, NCLASS), lambda i: (0, 0)),
        ],
        out_specs=[
            pl.BlockSpec((NC, MROW_BLK, HALF), lambda i: (0, i, 0)),
            pl.BlockSpec((NC, MROW_BLK, HALF), lambda i: (0, i, 0)),
        ],
        out_shape=[
            jax.ShapeDtypeStruct((NC, N, HALF), jnp.bfloat16),
            jax.ShapeDtypeStruct((NC, N, HALF), jnp.bfloat16),
        ],
    )(x, W1, b1.reshape(1, NCLASS), W2, b2.reshape(1, NCLASS))

    row = edge_index[0].astype(jnp.int32)
    col = edge_index[1].astype(jnp.int32)
    pad = NS * EP_TILE - E
    row_p = jnp.pad(row, (0, pad)).reshape(NS, NCH, CHUNK)
    col_p = jnp.pad(col, (0, pad)).reshape(NS, NCH, CHUNK)
    val_p = jnp.pad(values.astype(jnp.float32), (0, pad)).reshape(NS, NCH, CHUNK)

    z0p = jnp.pad(z0h, ((0, 0), (0, NP_ - N), (0, 0)))
    h0p = jnp.pad(h0a, ((0, 0), (0, NP_ - N), (0, 0)))

    mesh = plsc.VectorSubcoreMesh(core_axis_name="c", subcore_axis_name="s")
    zf = pl.kernel(
        _sc_body,
        out_type=jax.ShapeDtypeStruct((NC, NP_, HALF), jnp.bfloat16),
        mesh=mesh,
        compiler_params=pltpu.CompilerParams(use_tc_tiling_on_sc=False,
                                             needs_layout_passes=False),
        scratch_types=[
            pltpu.VMEM((NCH, CHUNK), jnp.int32),
            pltpu.VMEM((NCH, CHUNK), jnp.int32),
            pltpu.VMEM((NCH, CHUNK), jnp.float32),
            pltpu.VMEM((CHUNK, HALF), jnp.bfloat16),
            pltpu.VMEM((CHUNK, HALF), jnp.bfloat16),
            pltpu.VMEM((CHUNK, HALF), jnp.bfloat16),
            pltpu.VMEM((CHUNK, HALF), jnp.bfloat16),
            pltpu.VMEM_SHARED((NP_, HALF), jnp.bfloat16),
            pltpu.VMEM_SHARED((NP_, HALF), jnp.bfloat16),
            pltpu.SemaphoreType.DMA,
            pltpu.SemaphoreType.DMA,
            pltpu.SemaphoreType.DMA,
            pltpu.SemaphoreType.DMA,
        ],
    )(z0p, h0p, col_p, row_p, val_p)

    zcat = jnp.concatenate([zf[0, :N], zf[1, :N]], axis=1)

    out = pl.pallas_call(
        _lsm_body,
        grid=(N // MROW_BLK,),
        in_specs=[pl.BlockSpec((MROW_BLK, NCLASS), lambda i: (i, 0))],
        out_specs=pl.BlockSpec((MROW_BLK, NCLASS), lambda i: (i, 0)),
        out_shape=jax.ShapeDtypeStruct((N, NCLASS), jnp.float32),
    )(zcat)
    return out
